# baseline (device time: 358641 ns/iter reference)
import os

import jax
import jax.numpy as jnp
from jax import lax
from jax.experimental import pallas as pl
from jax.experimental.pallas import tpu as pltpu

N_DEV = 8

_INTERPRET = (
    pltpu.InterpretParams(detect_races=True, dma_execution_mode="on_wait")
    if os.environ.get("KERNEL_INTERPRET") == "1"
    else False
)

_MESH = pl.DeviceIdType.MESH


def kernel(x):
    m_per, n = x.shape
    ch = m_per // N_DEV
    ch2 = ch // 2
    ch4 = ch // 4

    n_q = 4

    def body(x_ref, out_ref, *scr):
        bufs = scr[0:4]
        stgs = scr[4:8]
        rs_snd = [scr[8 + 5 * q + 0] for q in range(n_q)]
        rs_rcv = [scr[8 + 5 * q + 1] for q in range(n_q)]
        ag_snd = [scr[8 + 5 * q + 2] for q in range(n_q)]
        ag_rcv = [scr[8 + 5 * q + 3] for q in range(n_q)]
        stg_sem = [scr[8 + 5 * q + 4] for q in range(n_q)]
        cred_rs = scr[28:32]
        cred_ag = scr[32:36]

        d = lax.axis_index("i")
        r = jnp.where(d < 4, d, 11 - d)
        nxt = jnp.where(d < 4, jnp.where(d == 3, 7, d + 1), jnp.where(d == 4, 0, d - 1))
        prv = jnp.where(d < 4, jnp.where(d == 0, 4, d - 1), jnp.where(d == 7, 3, d + 1))

        sub = [
            (nxt, prv, -1, 0 * ch4),
            (prv, nxt, +1, ch2),
            (nxt, prv, -1, 1 * ch4),
            (prv, nxt, +1, ch2 + ch4),
        ]

        barrier = pltpu.get_barrier_semaphore()
        for nbr in (nxt, prv):
            pl.semaphore_signal(barrier, inc=1, device_id=(nbr,), device_id_type=_MESH)
        pl.semaphore_wait(barrier, 2)

        def stage(q, s):
            _, _, sign, off = sub[q]
            c = (r + sign * s) % N_DEV
            cp = pltpu.make_async_copy(
                x_ref.at[pl.ds(c * ch + off, ch4), :],
                stgs[q].at[s % 2],
                stg_sem[q].at[s % 2],
            )
            cp.start()
            return cp

        def rs_start(q, s):
            tgt, _, _, _ = sub[q]
            src = stgs[q].at[0] if s == 0 else bufs[q].at[(s - 1) % 3]
            rd = pltpu.make_async_remote_copy(
                src_ref=src,
                dst_ref=bufs[q].at[s % 3],
                send_sem=rs_snd[q].at[s % 3],
                recv_sem=rs_rcv[q].at[s % 3],
                device_id=(tgt,),
                device_id_type=_MESH,
            )
            rd.start()
            return rd

        pend_stg = [[stage(q, 0), stage(q, 1)] for q in range(n_q)]
        rs_desc = [[] for _ in range(n_q)]
        for q in range(n_q):
            pend_stg[q][0].wait()
            rs_desc[q].append(rs_start(q, 0))

        for s in range(1, N_DEV - 1):
            for q in range(n_q):
                tgt, src_dev, _, _ = sub[q]
                rs_desc[q][s - 1].wait_recv()
                pend_stg[q][s].wait()
                bufs[q][(s - 1) % 3] = bufs[q][(s - 1) % 3] + stgs[q][s % 2]
                rs_desc[q][s - 1].wait_send()
                pend_stg[q].append(stage(q, s + 1))
                if 2 <= s <= 5:
                    pl.semaphore_signal(
                        cred_rs[q], inc=1, device_id=(src_dev,), device_id_type=_MESH
                    )
                if s >= 3:
                    pl.semaphore_wait(cred_rs[q], 1)
                rs_desc[q].append(rs_start(q, s))

        for q in range(n_q):
            _, _, sign, off = sub[q]
            rs_desc[q][N_DEV - 2].wait_recv()
            pend_stg[q][N_DEV - 1].wait()
            red = (r - sign) % N_DEV
            fin = (N_DEV - 2) % 3
            out_ref[pl.ds(red * ch + off, ch4), :] = bufs[q][fin] + stgs[q][1]
            rs_desc[q][N_DEV - 2].wait_send()

        def ag_start(q, t):
            tgt, _, sign, off = sub[q]
            c = (r + sign * (t - 1)) % N_DEV
            rd = pltpu.make_async_remote_copy(
                src_ref=out_ref.at[pl.ds(c * ch + off, ch4), :],
                dst_ref=out_ref.at[pl.ds(c * ch + off, ch4), :],
                send_sem=ag_snd[q].at[t % 3],
                recv_sem=ag_rcv[q].at[t % 3],
                device_id=(tgt,),
                device_id_type=_MESH,
            )
            rd.start()
            return rd

        ag_desc = [[] for _ in range(n_q)]
        for q in range(n_q):
            ag_desc[q].append(ag_start(q, 0))

        for t in range(1, N_DEV - 1):
            for q in range(n_q):
                _, src_dev, _, _ = sub[q]
                ag_desc[q][t - 1].wait_recv()
                ag_desc[q][t - 1].wait_send()
                if 1 <= t <= 4:
                    pl.semaphore_signal(
                        cred_ag[q], inc=1, device_id=(src_dev,), device_id_type=_MESH
                    )
                if t >= 3:
                    pl.semaphore_wait(cred_ag[q], 1)
                ag_desc[q].append(ag_start(q, t))

        for q in range(n_q):
            ag_desc[q][N_DEV - 2].wait_recv()
            ag_desc[q][N_DEV - 2].wait_send()

    scratch = (
        [pltpu.VMEM((3, ch4, n), x.dtype) for _ in range(4)]
        + [pltpu.VMEM((2, ch4, n), x.dtype) for _ in range(4)]
        + [
            sem
            for _ in range(4)
            for sem in (
                pltpu.SemaphoreType.DMA((3,)),
                pltpu.SemaphoreType.DMA((3,)),
                pltpu.SemaphoreType.DMA((3,)),
                pltpu.SemaphoreType.DMA((3,)),
                pltpu.SemaphoreType.DMA((2,)),
            )
        ]
        + [pltpu.SemaphoreType.REGULAR for _ in range(8)]
    )
    return pl.pallas_call(
        body,
        out_shape=jax.ShapeDtypeStruct((m_per, n), x.dtype),
        in_specs=[pl.BlockSpec(memory_space=pl.ANY)],
        out_specs=pl.BlockSpec(memory_space=pltpu.MemorySpace.VMEM),
        scratch_shapes=scratch,
        compiler_params=pltpu.CompilerParams(
            collective_id=0, vmem_limit_bytes=100 * 1024 * 1024
        ),
        interpret=_INTERPRET,
    )(x)


# device time: 264605 ns/iter; 1.3554x vs baseline; 1.3554x over previous
import os

import jax
import jax.numpy as jnp
from jax import lax
from jax.experimental import pallas as pl
from jax.experimental.pallas import tpu as pltpu

N_DEV = 8

_INTERPRET = (
    pltpu.InterpretParams(detect_races=True, dma_execution_mode="on_wait")
    if os.environ.get("KERNEL_INTERPRET") == "1"
    else False
)

_MESH = pl.DeviceIdType.MESH


def kernel(x):
    m_per, n = x.shape

    if m_per % 64 == 0 and (m_per // 64) % 3 != 0:
        u = m_per // 64
        r0 = (u // 3 + (1 if u % 3 > 0 else 0)) * 64
        r1 = (u // 3 + (1 if u % 3 > 1 else 0)) * 64
        r2 = m_per - r0 - r1
        bands = [r0, r1, r2]
    else:
        bands = [m_per // 3 * 1] * 3
        bands[2] = m_per - bands[0] - bands[1]
    base0 = [0, bands[0], bands[0] + bands[1]]

    def body(x_ref, out_ref, rcv0, rcv1, rcv2, *sems):
        rcvs = [rcv0, rcv1, rcv2]
        snd_sems = sems[0:3]
        rcv_sems = sems[3:6]
        ini_sems = sems[6:9]

        d = lax.axis_index("i")
        r4 = d % 4
        cz = d // 4
        cy = r4 // 2
        cx = (r4 % 2) ^ (r4 // 2)

        def pos(px, py, pz):
            return pz * 4 + (py * 2 + (px ^ py))

        partner = {
            "x": pos(1 - cx, cy, cz),
            "y": pos(cx, 1 - cy, cz),
            "z": pos(cx, cy, 1 - cz),
        }
        mybit = {"x": cx, "y": cy, "z": cz}
        order = [("x", "y", "z"), ("y", "z", "x"), ("z", "x", "y")]

        barrier = pltpu.get_barrier_semaphore()
        for dim in ("x", "y", "z"):
            pl.semaphore_signal(
                barrier, inc=1, device_id=(partner[dim],), device_id_type=_MESH
            )
        pl.semaphore_wait(barrier, 3)

        def rcv_off(p, k):
            R = bands[p]
            return 0 if k == 0 else (R // 2 if k == 1 else 3 * R // 4)

        inits = []
        sends = []
        for p in range(3):
            R = bands[p]
            dim = order[p][0]
            b = mybit[dim]
            cp = pltpu.make_async_copy(
                x_ref.at[pl.ds(base0[p] + b * (R // 2), R // 2), :],
                out_ref.at[pl.ds(base0[p], R // 2), :],
                ini_sems[p].at[0],
            )
            cp.start()
            inits.append(cp)
            rd = pltpu.make_async_remote_copy(
                src_ref=x_ref.at[pl.ds(base0[p] + (1 - b) * (R // 2), R // 2), :],
                dst_ref=rcvs[p].at[pl.ds(0, R // 2), :],
                send_sem=snd_sems[p].at[0],
                recv_sem=rcv_sems[p].at[0],
                device_id=(partner[dim],),
                device_id_type=_MESH,
            )
            rd.start()
            sends.append(rd)

        koffs = []
        for p in range(3):
            R = bands[p]
            sends[p].wait_recv()
            inits[p].wait()
            out_ref[pl.ds(base0[p], R // 2), :] = (
                out_ref[pl.ds(base0[p], R // 2), :] + rcvs[p][pl.ds(0, R // 2), :]
            )
            koffs.append(jnp.int32(0))

        for k in (1, 2):
            prev_sends = sends
            sends = []
            for p in range(3):
                R = bands[p]
                L = R // (2**k)
                dim = order[p][k]
                b = mybit[dim]
                prev_sends[p].wait_send()
                rd = pltpu.make_async_remote_copy(
                    src_ref=out_ref.at[
                        pl.ds(base0[p] + koffs[p] + (1 - b) * (L // 2), L // 2), :
                    ],
                    dst_ref=rcvs[p].at[pl.ds(rcv_off(p, k), L // 2), :],
                    send_sem=snd_sems[p].at[k],
                    recv_sem=rcv_sems[p].at[k],
                    device_id=(partner[dim],),
                    device_id_type=_MESH,
                )
                rd.start()
                sends.append(rd)
                koffs[p] = koffs[p] + b * (L // 2)
            for p in range(3):
                R = bands[p]
                L = R // (2**k)
                sends[p].wait_recv()
                out_ref[pl.ds(base0[p] + koffs[p], L // 2), :] = (
                    out_ref[pl.ds(base0[p] + koffs[p], L // 2), :]
                    + rcvs[p][pl.ds(rcv_off(p, k), L // 2), :]
                )

        gbase = []
        for p in range(3):
            R = bands[p]
            g = base0[p] + koffs[p] + mybit[order[p][0]] * (R // 2)
            gbase.append(g)
            sends[p].wait_send()
            out_ref[pl.ds(g, R // 8), :] = out_ref[pl.ds(base0[p] + koffs[p], R // 8), :]

        for j, k in enumerate((2, 1, 0)):
            prev_sends = sends
            sends = []
            for p in range(3):
                R = bands[p]
                S = R // (2 ** (k + 1))
                dim = order[p][k]
                if j > 0:
                    prev_sends[p].wait_send()
                rd = pltpu.make_async_remote_copy(
                    src_ref=out_ref.at[pl.ds(gbase[p], S), :],
                    dst_ref=out_ref.at[pl.ds(gbase[p], S), :],
                    send_sem=snd_sems[p].at[3 + j],
                    recv_sem=rcv_sems[p].at[3 + j],
                    device_id=(partner[dim],),
                    device_id_type=_MESH,
                )
                rd.start()
                sends.append(rd)
                gbase[p] = gbase[p] - mybit[dim] * S
            for p in range(3):
                sends[p].wait_recv()
        for p in range(3):
            sends[p].wait_send()

    scratch = (
        [pltpu.VMEM((bands[p] * 7 // 8, n), x.dtype) for p in range(3)]
        + [pltpu.SemaphoreType.DMA((6,)) for _ in range(3)]
        + [pltpu.SemaphoreType.DMA((6,)) for _ in range(3)]
        + [pltpu.SemaphoreType.DMA((1,)) for _ in range(3)]
    )
    return pl.pallas_call(
        body,
        out_shape=jax.ShapeDtypeStruct((m_per, n), x.dtype),
        in_specs=[pl.BlockSpec(memory_space=pl.ANY)],
        out_specs=pl.BlockSpec(memory_space=pltpu.MemorySpace.VMEM),
        scratch_shapes=scratch,
        compiler_params=pltpu.CompilerParams(
            collective_id=0, vmem_limit_bytes=100 * 1024 * 1024
        ),
        interpret=_INTERPRET,
    )(x)


# device time: 263789 ns/iter; 1.3596x vs baseline; 1.0031x over previous
import os

import jax
import jax.numpy as jnp
from jax import lax
from jax.experimental import pallas as pl
from jax.experimental.pallas import tpu as pltpu

N_DEV = 8

_INTERPRET = (
    pltpu.InterpretParams(detect_races=True, dma_execution_mode="on_wait")
    if os.environ.get("KERNEL_INTERPRET") == "1"
    else False
)

_MESH = pl.DeviceIdType.MESH


def kernel(x):
    m_per, n = x.shape

    if m_per % 64 == 0 and (m_per // 64) % 3 != 0:
        u = m_per // 64
        r0 = (u // 3 + (1 if u % 3 > 0 else 0)) * 64
        r1 = (u // 3 + (1 if u % 3 > 1 else 0)) * 64
        r2 = m_per - r0 - r1
        bands = [r0, r1, r2]
    else:
        bands = [m_per // 3 * 1] * 3
        bands[2] = m_per - bands[0] - bands[1]
    base0 = [0, bands[0], bands[0] + bands[1]]

    def body(x_ref, out_ref, rcv0, rcv1, rcv2, *sems):
        rcvs = [rcv0, rcv1, rcv2]
        snd_sems = sems[0:3]
        rcv_sems = sems[3:6]
        ini_sems = sems[6:9]

        d = lax.axis_index("i")
        r4 = d % 4
        cz = d // 4
        cy = r4 // 2
        cx = (r4 % 2) ^ (r4 // 2)

        def pos(px, py, pz):
            return pz * 4 + (py * 2 + (px ^ py))

        partner = {
            "x": pos(1 - cx, cy, cz),
            "y": pos(cx, 1 - cy, cz),
            "z": pos(cx, cy, 1 - cz),
        }
        mybit = {"x": cx, "y": cy, "z": cz}
        order = [("x", "y", "z"), ("y", "z", "x"), ("z", "x", "y")]

        barrier = pltpu.get_barrier_semaphore()
        for dim in ("x", "y", "z"):
            pl.semaphore_signal(
                barrier, inc=1, device_id=(partner[dim],), device_id_type=_MESH
            )
        pl.semaphore_wait(barrier, 3)

        def rcv_off(p, k):
            R = bands[p]
            return 0 if k == 0 else (R // 2 if k == 1 else 3 * R // 4)

        inits = []
        sends = []
        sends_b = []
        for p in range(3):
            R = bands[p]
            Q = R // 4
            dim = order[p][0]
            b = mybit[dim]
            cp = pltpu.make_async_copy(
                x_ref.at[pl.ds(base0[p] + b * (R // 2), R // 2), :],
                out_ref.at[pl.ds(base0[p], R // 2), :],
                ini_sems[p].at[0],
            )
            cp.start()
            inits.append(cp)
            half = base0[p] + (1 - b) * (R // 2)
            rd = pltpu.make_async_remote_copy(
                src_ref=x_ref.at[pl.ds(half, Q), :],
                dst_ref=rcvs[p].at[pl.ds(0, Q), :],
                send_sem=snd_sems[p].at[0],
                recv_sem=rcv_sems[p].at[0],
                device_id=(partner[dim],),
                device_id_type=_MESH,
            )
            rd.start()
            sends.append(rd)
            rb = pltpu.make_async_remote_copy(
                src_ref=x_ref.at[pl.ds(half + Q, Q), :],
                dst_ref=rcvs[p].at[pl.ds(Q, Q), :],
                send_sem=snd_sems[p].at[6],
                recv_sem=rcv_sems[p].at[6],
                device_id=(partner[dim],),
                device_id_type=_MESH,
            )
            rb.start()
            sends_b.append(rb)

        koffs = []
        for p in range(3):
            R = bands[p]
            Q = R // 4
            sends[p].wait_recv()
            inits[p].wait()
            out_ref[pl.ds(base0[p], Q), :] = (
                out_ref[pl.ds(base0[p], Q), :] + rcvs[p][pl.ds(0, Q), :]
            )
        for p in range(3):
            R = bands[p]
            Q = R // 4
            sends_b[p].wait_recv()
            out_ref[pl.ds(base0[p] + Q, Q), :] = (
                out_ref[pl.ds(base0[p] + Q, Q), :] + rcvs[p][pl.ds(Q, Q), :]
            )
            koffs.append(jnp.int32(0))

        for k in (1, 2):
            prev_sends = sends
            sends = []
            for p in range(3):
                R = bands[p]
                L = R // (2**k)
                dim = order[p][k]
                b = mybit[dim]
                prev_sends[p].wait_send()
                if k == 1:
                    sends_b[p].wait_send()
                rd = pltpu.make_async_remote_copy(
                    src_ref=out_ref.at[
                        pl.ds(base0[p] + koffs[p] + (1 - b) * (L // 2), L // 2), :
                    ],
                    dst_ref=rcvs[p].at[pl.ds(rcv_off(p, k), L // 2), :],
                    send_sem=snd_sems[p].at[k],
                    recv_sem=rcv_sems[p].at[k],
                    device_id=(partner[dim],),
                    device_id_type=_MESH,
                )
                rd.start()
                sends.append(rd)
                koffs[p] = koffs[p] + b * (L // 2)
            for p in range(3):
                R = bands[p]
                L = R // (2**k)
                sends[p].wait_recv()
                out_ref[pl.ds(base0[p] + koffs[p], L // 2), :] = (
                    out_ref[pl.ds(base0[p] + koffs[p], L // 2), :]
                    + rcvs[p][pl.ds(rcv_off(p, k), L // 2), :]
                )

        gbase = []
        for p in range(3):
            R = bands[p]
            g = base0[p] + koffs[p] + mybit[order[p][0]] * (R // 2)
            gbase.append(g)
            sends[p].wait_send()
            out_ref[pl.ds(g, R // 8), :] = out_ref[pl.ds(base0[p] + koffs[p], R // 8), :]

        for j, k in enumerate((2, 1, 0)):
            prev_sends = sends
            sends = []
            for p in range(3):
                R = bands[p]
                S = R // (2 ** (k + 1))
                dim = order[p][k]
                if j > 0:
                    prev_sends[p].wait_send()
                rd = pltpu.make_async_remote_copy(
                    src_ref=out_ref.at[pl.ds(gbase[p], S), :],
                    dst_ref=out_ref.at[pl.ds(gbase[p], S), :],
                    send_sem=snd_sems[p].at[3 + j],
                    recv_sem=rcv_sems[p].at[3 + j],
                    device_id=(partner[dim],),
                    device_id_type=_MESH,
                )
                rd.start()
                sends.append(rd)
                gbase[p] = gbase[p] - mybit[dim] * S
            for p in range(3):
                sends[p].wait_recv()
        for p in range(3):
            sends[p].wait_send()

    scratch = (
        [pltpu.VMEM((bands[p] * 7 // 8, n), x.dtype) for p in range(3)]
        + [pltpu.SemaphoreType.DMA((8,)) for _ in range(3)]
        + [pltpu.SemaphoreType.DMA((8,)) for _ in range(3)]
        + [pltpu.SemaphoreType.DMA((1,)) for _ in range(3)]
    )
    return pl.pallas_call(
        body,
        out_shape=jax.ShapeDtypeStruct((m_per, n), x.dtype),
        in_specs=[pl.BlockSpec(memory_space=pl.ANY)],
        out_specs=pl.BlockSpec(memory_space=pltpu.MemorySpace.VMEM),
        scratch_shapes=scratch,
        compiler_params=pltpu.CompilerParams(
            collective_id=0, vmem_limit_bytes=100 * 1024 * 1024
        ),
        interpret=_INTERPRET,
    )(x)
